# Initial kernel scaffold; baseline (speedup 1.0000x reference)
import functools

import jax
import jax.numpy as jnp
from jax import lax
from jax.experimental import pallas as pl
from jax.experimental.pallas import tpu as pltpu
from jax.experimental.pallas import tpu_sc as plsc

N = 100000
D = 128
S = 1024
L = 16
NC = 2
NS = 16
NW = NC * NS
CHUNK = 256
NFULL = N // CHUNK
TAIL = N - NFULL * CHUNK
ACC = S * L

_mesh = plsc.VectorSubcoreMesh(core_axis_name="c", subcore_axis_name="s")


def _wid():
    return lax.axis_index("s") * NC + lax.axis_index("c")


@functools.partial(
    pl.kernel,
    mesh=_mesh,
    out_type=[
        jax.ShapeDtypeStruct((NW * ACC,), jnp.float32),
        jax.ShapeDtypeStruct((NW * ACC,), jnp.float32),
    ],
    scratch_types=[
        pltpu.VMEM((CHUNK * D,), jnp.float32),
        pltpu.VMEM((CHUNK,), jnp.int32),
        pltpu.VMEM((D,), jnp.float32),
        pltpu.VMEM((L,), jnp.int32),
        pltpu.VMEM((ACC,), jnp.float32),
        pltpu.VMEM((ACC,), jnp.float32),
    ],
)
def _phase1(x_hbm, b_hbm, w_hbm, a_hbm, c_hbm, xbuf, bbuf, wbuf, tmp, acc, cnt):
    wid = _wid()
    iota = lax.iota(jnp.int32, L)
    zero16 = jnp.zeros((L,), jnp.float32)
    ones16 = jnp.ones((L,), jnp.float32)

    pltpu.sync_copy(w_hbm, wbuf)
    wv = [wbuf[pl.ds(16 * c, 16)] for c in range(8)]

    def zbody(i, _):
        acc[pl.ds(i * 16, 16)] = zero16
        cnt[pl.ds(i * 16, 16)] = zero16
        return 0
    lax.fori_loop(0, S, zbody, 0)

    def make_chunk(ngroups):
        nrows = ngroups * L

        def do_chunk(rowbase):
            pltpu.sync_copy(x_hbm.at[pl.ds(rowbase * D, nrows * D)],
                            xbuf.at[pl.ds(0, nrows * D)])
            pltpu.sync_copy(b_hbm.at[pl.ds(rowbase, nrows)],
                            bbuf.at[pl.ds(0, nrows)])

            def gbody(g, _):
                r0 = g * L
                bv = bbuf[pl.ds(r0, 16)]
                idxb = bv * 16
                plsc.addupdate_scatter(cnt, [idxb + iota], ones16)
                tmp[pl.ds(0, 16)] = idxb
                for i in range(L):
                    bs = plsc.load_gather(tmp, [jnp.full((L,), i, jnp.int32)])
                    xoff = (r0 + i) * D
                    y = xbuf[pl.ds(xoff, 16)] * wv[0]
                    for c in range(1, 8):
                        y = y + xbuf[pl.ds(xoff + c * 16, 16)] * wv[c]
                    plsc.addupdate_scatter(acc, [bs + iota], y)
                return 0
            lax.fori_loop(0, ngroups, gbody, 0)
        return do_chunk

    full_chunk = make_chunk(CHUNK // L)
    trips = jnp.where(wid < NFULL % NW, NFULL // NW + 1, NFULL // NW)

    def cbody(k, _):
        full_chunk((wid + k * NW) * CHUNK)
        return 0
    lax.fori_loop(0, trips, cbody, 0)

    @pl.when(wid == NW - 1)
    def _():
        make_chunk(TAIL // L)(NFULL * CHUNK)

    pltpu.sync_copy(acc, a_hbm.at[pl.ds(wid * ACC, ACC)])
    pltpu.sync_copy(cnt, c_hbm.at[pl.ds(wid * ACC, ACC)])


SEGW = S // NW
WIN = SEGW * L


@functools.partial(
    pl.kernel,
    mesh=_mesh,
    out_type=jax.ShapeDtypeStruct((S,), jnp.float32),
    scratch_types=[
        pltpu.VMEM((WIN,), jnp.float32),
        pltpu.VMEM((WIN,), jnp.float32),
        pltpu.VMEM((WIN,), jnp.float32),
        pltpu.VMEM((WIN,), jnp.float32),
        pltpu.VMEM((L,), jnp.float32),
        pltpu.VMEM((SEGW,), jnp.float32),
    ],
)
def _phase2(a_hbm, c_hbm, bias_hbm, out_hbm, sa, sc, wa, wc, bbuf, outv):
    wid = _wid()
    sb = wid * WIN
    zero16 = jnp.zeros((L,), jnp.float32)

    def zbody(i, _):
        sa[pl.ds(i * 16, 16)] = zero16
        sc[pl.ds(i * 16, 16)] = zero16
        return 0
    lax.fori_loop(0, SEGW, zbody, 0)

    def pbody(p, _):
        pltpu.sync_copy(a_hbm.at[pl.ds(p * ACC + sb, WIN)], wa)
        pltpu.sync_copy(c_hbm.at[pl.ds(p * ACC + sb, WIN)], wc)
        for v in range(SEGW):
            sl = pl.ds(v * 16, 16)
            sa[sl] = sa[sl] + wa[sl]
            sc[sl] = sc[sl] + wc[sl]
        return 0
    lax.fori_loop(0, NW, pbody, 0)

    pltpu.sync_copy(bias_hbm, bbuf)
    bv = bbuf[pl.ds(0, 16)]
    iota16 = lax.iota(jnp.int32, L) * 16
    for g in range(SEGW // L):
        ta = jnp.zeros((L,), jnp.float32)
        tc = jnp.zeros((L,), jnp.float32)
        for l in range(L):
            idx = iota16 + (g * 256 + l)
            ta = ta + plsc.load_gather(sa, [idx])
            tc = tc + plsc.load_gather(sc, [idx])
        outv[pl.ds(g * 16, 16)] = ta / jnp.maximum(tc, 1.0) + bv
    pltpu.sync_copy(outv, out_hbm.at[pl.ds(wid * SEGW, SEGW)])


def kernel(x, batch, W, b):
    x1 = x.reshape(-1)
    bi = batch.astype(jnp.int32)
    wv = W.reshape(D).astype(jnp.float32)
    b16 = jnp.broadcast_to(b.astype(jnp.float32), (L,))
    a, c = _phase1(x1, bi, wv)
    return _phase2(a, c, b16)


# speedup vs baseline: 3.3477x; 3.3477x over previous
import functools

import jax
import jax.numpy as jnp
from jax import lax
from jax.experimental import pallas as pl
from jax.experimental.pallas import tpu as pltpu
from jax.experimental.pallas import tpu_sc as plsc

N = 100000
D = 128
S = 1024
L = 16
NC = 2
NS = 16
NW = NC * NS
CHUNK = 256
NFULL = N // CHUNK
TAIL = N - NFULL * CHUNK
ACC = S * L

_mesh = plsc.VectorSubcoreMesh(core_axis_name="c", subcore_axis_name="s")
_params = pltpu.CompilerParams(needs_layout_passes=False)


def _wid():
    return lax.axis_index("s") * NC + lax.axis_index("c")


@functools.partial(
    pl.kernel,
    mesh=_mesh,
    out_type=[
        jax.ShapeDtypeStruct((NW * ACC,), jnp.float32),
        jax.ShapeDtypeStruct((NW * ACC,), jnp.float32),
    ],
    scratch_types=[
        pltpu.VMEM((CHUNK * D,), jnp.float32),
        pltpu.VMEM((CHUNK,), jnp.int32),
        pltpu.VMEM((D,), jnp.float32),
        pltpu.VMEM((ACC,), jnp.float32),
        pltpu.VMEM((ACC,), jnp.float32),
    ],
    compiler_params=_params,
)
def _phase1(x_hbm, b_hbm, w_hbm, a_hbm, c_hbm, xbuf, bbuf, wbuf, acc, cnt):
    wid = _wid()
    iota = lax.iota(jnp.int32, L)
    zero16 = jnp.zeros((L,), jnp.float32)
    ones16 = jnp.ones((L,), jnp.float32)

    pltpu.sync_copy(w_hbm, wbuf)
    wv = [wbuf[pl.ds(16 * c, 16)] for c in range(8)]

    def zbody(i, _):
        acc[pl.ds(i * 16, 16)] = zero16
        cnt[pl.ds(i * 16, 16)] = zero16
        return 0
    lax.fori_loop(0, S, zbody, 0)

    def make_chunk(ngroups):
        nrows = ngroups * L

        def do_chunk(rowbase):
            pltpu.sync_copy(x_hbm.at[pl.ds(rowbase * D, nrows * D)],
                            xbuf.at[pl.ds(0, nrows * D)])
            pltpu.sync_copy(b_hbm.at[pl.ds(rowbase, nrows)],
                            bbuf.at[pl.ds(0, nrows)])

            def gbody(g, _):
                r0 = g * L
                bv = bbuf[pl.ds(r0, 16)]
                idxb = bv * 16
                plsc.addupdate_scatter(cnt, [idxb + iota], ones16)
                for i in range(L):
                    bs = jnp.take_along_axis(
                        idxb, jnp.full((L,), i, jnp.int32), axis=0,
                        mode="promise_in_bounds")
                    xoff = (r0 + i) * D
                    y = xbuf[pl.ds(xoff, 16)] * wv[0]
                    for c in range(1, 8):
                        y = y + xbuf[pl.ds(xoff + c * 16, 16)] * wv[c]
                    plsc.addupdate_scatter(acc, [bs + iota], y)
                return 0
            lax.fori_loop(0, ngroups, gbody, 0)
        return do_chunk

    full_chunk = make_chunk(CHUNK // L)
    trips = jnp.where(wid < NFULL % NW, NFULL // NW + 1, NFULL // NW)

    def cbody(k, _):
        full_chunk((wid + k * NW) * CHUNK)
        return 0
    lax.fori_loop(0, trips, cbody, 0)

    @pl.when(wid == NW - 1)
    def _():
        make_chunk(TAIL // L)(NFULL * CHUNK)

    pltpu.sync_copy(acc, a_hbm.at[pl.ds(wid * ACC, ACC)])
    pltpu.sync_copy(cnt, c_hbm.at[pl.ds(wid * ACC, ACC)])


SEGW = S // NW
WIN = SEGW * L


@functools.partial(
    pl.kernel,
    mesh=_mesh,
    out_type=jax.ShapeDtypeStruct((S,), jnp.float32),
    scratch_types=[
        pltpu.VMEM((WIN,), jnp.float32),
        pltpu.VMEM((WIN,), jnp.float32),
        pltpu.VMEM((WIN,), jnp.float32),
        pltpu.VMEM((WIN,), jnp.float32),
        pltpu.VMEM((L,), jnp.float32),
        pltpu.VMEM((SEGW,), jnp.float32),
    ],
    compiler_params=_params,
)
def _phase2(a_hbm, c_hbm, bias_hbm, out_hbm, sa, sc, wa, wc, bbuf, outv):
    wid = _wid()
    sb = wid * WIN
    zero16 = jnp.zeros((L,), jnp.float32)

    def zbody(i, _):
        sa[pl.ds(i * 16, 16)] = zero16
        sc[pl.ds(i * 16, 16)] = zero16
        return 0
    lax.fori_loop(0, SEGW, zbody, 0)

    def pbody(p, _):
        pltpu.sync_copy(a_hbm.at[pl.ds(p * ACC + sb, WIN)], wa)
        pltpu.sync_copy(c_hbm.at[pl.ds(p * ACC + sb, WIN)], wc)
        for v in range(SEGW):
            sl = pl.ds(v * 16, 16)
            sa[sl] = sa[sl] + wa[sl]
            sc[sl] = sc[sl] + wc[sl]
        return 0
    lax.fori_loop(0, NW, pbody, 0)

    pltpu.sync_copy(bias_hbm, bbuf)
    bv = bbuf[pl.ds(0, 16)]
    iota16 = lax.iota(jnp.int32, L) * 16
    for g in range(SEGW // L):
        ta = jnp.zeros((L,), jnp.float32)
        tc = jnp.zeros((L,), jnp.float32)
        for l in range(L):
            idx = iota16 + (g * 256 + l)
            ta = ta + plsc.load_gather(sa, [idx])
            tc = tc + plsc.load_gather(sc, [idx])
        outv[pl.ds(g * 16, 16)] = ta / jnp.maximum(tc, 1.0) + bv
    pltpu.sync_copy(outv, out_hbm.at[pl.ds(wid * SEGW, SEGW)])


def kernel(x, batch, W, b):
    x1 = x.reshape(-1)
    bi = batch.astype(jnp.int32)
    wv = W.reshape(D).astype(jnp.float32)
    b16 = jnp.broadcast_to(b.astype(jnp.float32), (L,))
    a, c = _phase1(x1, bi, wv)
    return _phase2(a, c, b16)


# speedup vs baseline: 5.6415x; 1.6852x over previous
import functools

import jax
import jax.numpy as jnp
from jax import lax
from jax.experimental import pallas as pl
from jax.experimental.pallas import tpu as pltpu
from jax.experimental.pallas import tpu_sc as plsc

N = 100000
D = 128
S = 1024
L = 16
NC = 2
NS = 16
NW = NC * NS
CHUNK = 256
CD = CHUNK * D
NFULL = N // CHUNK
TAIL = N - NFULL * CHUNK
ACC = S * L

_mesh = plsc.VectorSubcoreMesh(core_axis_name="c", subcore_axis_name="s")
_params = pltpu.CompilerParams(needs_layout_passes=False)


def _wid():
    return lax.axis_index("s") * NC + lax.axis_index("c")


@functools.partial(
    pl.kernel,
    mesh=_mesh,
    out_type=[
        jax.ShapeDtypeStruct((NW * ACC,), jnp.float32),
        jax.ShapeDtypeStruct((NW * ACC,), jnp.float32),
    ],
    scratch_types=[
        pltpu.VMEM((2 * CD,), jnp.float32),
        pltpu.VMEM((2 * CHUNK,), jnp.int32),
        pltpu.VMEM((D,), jnp.float32),
        pltpu.VMEM((ACC,), jnp.float32),
        pltpu.VMEM((ACC,), jnp.float32),
        pltpu.SemaphoreType.DMA,
        pltpu.SemaphoreType.DMA,
    ],
    compiler_params=_params,
)
def _phase1(x_hbm, b_hbm, w_hbm, a_hbm, c_hbm,
            xbuf, bbuf, wbuf, acc, cnt, semx, semb):
    wid = _wid()
    iota = lax.iota(jnp.int32, L)
    zero16 = jnp.zeros((L,), jnp.float32)
    ones16 = jnp.ones((L,), jnp.float32)

    pltpu.sync_copy(w_hbm, wbuf)
    wv = [wbuf[pl.ds(16 * c, 16)] for c in range(8)]

    def zbody(i, _):
        acc[pl.ds(i * 16, 16)] = zero16
        cnt[pl.ds(i * 16, 16)] = zero16
        return 0
    lax.fori_loop(0, S, zbody, 0)

    trips = jnp.where(wid < NFULL % NW, NFULL // NW + 1, NFULL // NW)

    def chunk_rowbase(k):
        return (wid + k * NW) * CHUNK

    def start_dma(k, par):
        rb = chunk_rowbase(k)
        pltpu.async_copy(x_hbm.at[pl.ds(rb * D, CD)],
                         xbuf.at[pl.ds(par * CD, CD)], semx)
        pltpu.async_copy(b_hbm.at[pl.ds(rb, CHUNK)],
                         bbuf.at[pl.ds(par * CHUNK, CHUNK)], semb)

    def process(xoff0, boff0, ngroups):
        def gbody(g, _):
            r0 = g * L
            bv = bbuf[pl.ds(boff0 + r0, 16)]
            idxb = bv * 16
            plsc.addupdate_scatter(cnt, [idxb + iota], ones16)
            for i in range(L):
                bs = jnp.take_along_axis(
                    idxb, jnp.full((L,), i, jnp.int32), axis=0,
                    mode="promise_in_bounds")
                xoff = xoff0 + (r0 + i) * D
                y = xbuf[pl.ds(xoff, 16)] * wv[0]
                for c in range(1, 8):
                    y = y + xbuf[pl.ds(xoff + c * 16, 16)] * wv[c]
                plsc.addupdate_scatter(acc, [bs + iota], y)
            return 0
        lax.fori_loop(0, ngroups, gbody, 0)

    start_dma(0, 0)

    def cbody(k, _):
        par = lax.rem(k, 2)
        pltpu.make_async_copy(x_hbm.at[pl.ds(0, CD)],
                              xbuf.at[pl.ds(par * CD, CD)], semx).wait()
        pltpu.make_async_copy(b_hbm.at[pl.ds(0, CHUNK)],
                              bbuf.at[pl.ds(par * CHUNK, CHUNK)], semb).wait()

        @pl.when(k + 1 < trips)
        def _():
            start_dma(k + 1, 1 - par)

        process(par * CD, par * CHUNK, CHUNK // L)
        return 0
    lax.fori_loop(0, trips, cbody, 0)

    @pl.when(wid == NW - 1)
    def _():
        pltpu.sync_copy(x_hbm.at[pl.ds(NFULL * CD, TAIL * D)],
                        xbuf.at[pl.ds(0, TAIL * D)])
        pltpu.sync_copy(b_hbm.at[pl.ds(NFULL * CHUNK, TAIL)],
                        bbuf.at[pl.ds(0, TAIL)])
        process(0, 0, TAIL // L)

    pltpu.sync_copy(acc, a_hbm.at[pl.ds(wid * ACC, ACC)])
    pltpu.sync_copy(cnt, c_hbm.at[pl.ds(wid * ACC, ACC)])


SEGW = S // NW
WIN = SEGW * L


@functools.partial(
    pl.kernel,
    mesh=_mesh,
    out_type=jax.ShapeDtypeStruct((S,), jnp.float32),
    scratch_types=[
        pltpu.VMEM((NW * WIN,), jnp.float32),
        pltpu.VMEM((NW * WIN,), jnp.float32),
        pltpu.VMEM((L,), jnp.float32),
        pltpu.VMEM((SEGW,), jnp.float32),
        pltpu.SemaphoreType.DMA,
    ],
    compiler_params=_params,
)
def _phase2(a_hbm, c_hbm, bias_hbm, out_hbm, wa, wc, bbuf, outv, sem):
    wid = _wid()
    sb = wid * WIN

    for half in range(4):
        for j in range(16):
            p = (half * 16 + j) % NW
            if half < 2:
                pltpu.async_copy(a_hbm.at[pl.ds(p * ACC + sb, WIN)],
                                 wa.at[pl.ds(p * WIN, WIN)], sem)
            else:
                pltpu.async_copy(c_hbm.at[pl.ds(p * ACC + sb, WIN)],
                                 wc.at[pl.ds(p * WIN, WIN)], sem)
        for j in range(16):
            p = (half * 16 + j) % NW
            if half < 2:
                pltpu.make_async_copy(a_hbm.at[pl.ds(0, WIN)],
                                      wa.at[pl.ds(p * WIN, WIN)], sem).wait()
            else:
                pltpu.make_async_copy(c_hbm.at[pl.ds(0, WIN)],
                                      wc.at[pl.ds(p * WIN, WIN)], sem).wait()

    pltpu.sync_copy(bias_hbm, bbuf)
    bv = bbuf[pl.ds(0, 16)]
    iota16 = lax.iota(jnp.int32, L) * 16

    def vbody(v, _):
        off = v * 16
        sa = wa[pl.ds(off, 16)]
        sc = wc[pl.ds(off, 16)]
        for p in range(1, NW):
            sa = sa + wa[pl.ds(p * WIN + off, 16)]
            sc = sc + wc[pl.ds(p * WIN + off, 16)]
        wa[pl.ds(off, 16)] = sa
        wc[pl.ds(off, 16)] = sc
        return 0
    lax.fori_loop(0, SEGW, vbody, 0)

    for g in range(SEGW // L):
        ta = jnp.zeros((L,), jnp.float32)
        tc = jnp.zeros((L,), jnp.float32)
        for l in range(L):
            idx = iota16 + (g * 256 + l)
            ta = ta + plsc.load_gather(wa, [idx])
            tc = tc + plsc.load_gather(wc, [idx])
        outv[pl.ds(g * 16, 16)] = ta / jnp.maximum(tc, 1.0) + bv
    pltpu.sync_copy(outv, out_hbm.at[pl.ds(wid * SEGW, SEGW)])


def kernel(x, batch, W, b):
    x1 = x.reshape(-1)
    bi = batch.astype(jnp.int32)
    wv = W.reshape(D).astype(jnp.float32)
    b16 = jnp.broadcast_to(b.astype(jnp.float32), (L,))
    a, c = _phase1(x1, bi, wv)
    return _phase2(a, c, b16)


# speedup vs baseline: 5.8226x; 1.0321x over previous
import functools

import jax
import jax.numpy as jnp
from jax import lax
from jax.experimental import pallas as pl
from jax.experimental.pallas import tpu as pltpu
from jax.experimental.pallas import tpu_sc as plsc

N = 100000
D = 128
S = 1024
L = 16
NC = 2
NS = 16
NW = NC * NS
CHUNK = 256
CD = CHUNK * D
NFULL = N // CHUNK
TAIL = N - NFULL * CHUNK
ACC = S * L

_mesh = plsc.VectorSubcoreMesh(core_axis_name="c", subcore_axis_name="s")
_params = pltpu.CompilerParams(needs_layout_passes=False)


def _wid():
    return lax.axis_index("s") * NC + lax.axis_index("c")


@functools.partial(
    pl.kernel,
    mesh=_mesh,
    out_type=[
        jax.ShapeDtypeStruct((NW * ACC,), jnp.float32),
        jax.ShapeDtypeStruct((NW * ACC,), jnp.float32),
    ],
    scratch_types=[
        pltpu.VMEM((2 * CD,), jnp.float32),
        pltpu.VMEM((2 * CHUNK,), jnp.int32),
        pltpu.VMEM((D,), jnp.float32),
        pltpu.VMEM((ACC,), jnp.float32),
        pltpu.VMEM((ACC,), jnp.float32),
        pltpu.SemaphoreType.DMA,
        pltpu.SemaphoreType.DMA,
    ],
    compiler_params=_params,
)
def _phase1(x_hbm, b_hbm, w_hbm, a_hbm, c_hbm,
            xbuf, bbuf, wbuf, acc, cnt, semx, semb):
    wid = _wid()
    iota = lax.iota(jnp.int32, L)
    zero16 = jnp.zeros((L,), jnp.float32)
    ones16 = jnp.ones((L,), jnp.float32)

    pltpu.sync_copy(w_hbm, wbuf)
    wv = [wbuf[pl.ds(16 * c, 16)] for c in range(8)]

    def zbody(i, _):
        acc[pl.ds(i * 16, 16)] = zero16
        cnt[pl.ds(i * 16, 16)] = zero16
        return 0
    lax.fori_loop(0, S, zbody, 0)

    trips = jnp.where(wid < NFULL % NW, NFULL // NW + 1, NFULL // NW)

    def chunk_rowbase(k):
        return (wid + k * NW) * CHUNK

    def start_dma(k, par):
        rb = chunk_rowbase(k)
        pltpu.async_copy(x_hbm.at[pl.ds(rb * D, CD)],
                         xbuf.at[pl.ds(par * CD, CD)], semx)
        pltpu.async_copy(b_hbm.at[pl.ds(rb, CHUNK)],
                         bbuf.at[pl.ds(par * CHUNK, CHUNK)], semb)

    def process(xoff0, boff0, ngroups, unroll=2):
        def one_group(r0):
            bv = bbuf[pl.ds(boff0 + r0, 16)]
            idxb = bv * 16
            plsc.addupdate_scatter(cnt, [idxb + iota], ones16)
            for i in range(L):
                bs = jnp.take_along_axis(
                    idxb, jnp.full((L,), i, jnp.int32), axis=0,
                    mode="promise_in_bounds")
                xoff = xoff0 + (r0 + i) * D
                p = [xbuf[pl.ds(xoff + c * 16, 16)] * wv[c] for c in range(8)]
                y = ((p[0] + p[1]) + (p[2] + p[3])) + ((p[4] + p[5]) + (p[6] + p[7]))
                plsc.addupdate_scatter(acc, [bs + iota], y)

        def gbody(g, _):
            for u in range(unroll):
                one_group((g * unroll + u) * L)
            return 0
        assert ngroups % unroll == 0
        lax.fori_loop(0, ngroups // unroll, gbody, 0)

    start_dma(0, 0)

    def cbody(k, _):
        par = lax.rem(k, 2)
        pltpu.make_async_copy(x_hbm.at[pl.ds(0, CD)],
                              xbuf.at[pl.ds(par * CD, CD)], semx).wait()
        pltpu.make_async_copy(b_hbm.at[pl.ds(0, CHUNK)],
                              bbuf.at[pl.ds(par * CHUNK, CHUNK)], semb).wait()

        @pl.when(k + 1 < trips)
        def _():
            start_dma(k + 1, 1 - par)

        process(par * CD, par * CHUNK, CHUNK // L)
        return 0
    lax.fori_loop(0, trips, cbody, 0)

    @pl.when(wid == NW - 1)
    def _():
        pltpu.sync_copy(x_hbm.at[pl.ds(NFULL * CD, TAIL * D)],
                        xbuf.at[pl.ds(0, TAIL * D)])
        pltpu.sync_copy(b_hbm.at[pl.ds(NFULL * CHUNK, TAIL)],
                        bbuf.at[pl.ds(0, TAIL)])
        process(0, 0, TAIL // L)

    pltpu.sync_copy(acc, a_hbm.at[pl.ds(wid * ACC, ACC)])
    pltpu.sync_copy(cnt, c_hbm.at[pl.ds(wid * ACC, ACC)])


SEGW = S // NW
WIN = SEGW * L


@functools.partial(
    pl.kernel,
    mesh=_mesh,
    out_type=jax.ShapeDtypeStruct((S,), jnp.float32),
    scratch_types=[
        pltpu.VMEM((NW * WIN,), jnp.float32),
        pltpu.VMEM((NW * WIN,), jnp.float32),
        pltpu.VMEM((L,), jnp.float32),
        pltpu.VMEM((SEGW,), jnp.float32),
        pltpu.SemaphoreType.DMA,
    ],
    compiler_params=_params,
)
def _phase2(a_hbm, c_hbm, bias_hbm, out_hbm, wa, wc, bbuf, outv, sem):
    wid = _wid()
    sb = wid * WIN

    for half in range(4):
        for j in range(16):
            p = (half * 16 + j) % NW
            if half < 2:
                pltpu.async_copy(a_hbm.at[pl.ds(p * ACC + sb, WIN)],
                                 wa.at[pl.ds(p * WIN, WIN)], sem)
            else:
                pltpu.async_copy(c_hbm.at[pl.ds(p * ACC + sb, WIN)],
                                 wc.at[pl.ds(p * WIN, WIN)], sem)
        for j in range(16):
            p = (half * 16 + j) % NW
            if half < 2:
                pltpu.make_async_copy(a_hbm.at[pl.ds(0, WIN)],
                                      wa.at[pl.ds(p * WIN, WIN)], sem).wait()
            else:
                pltpu.make_async_copy(c_hbm.at[pl.ds(0, WIN)],
                                      wc.at[pl.ds(p * WIN, WIN)], sem).wait()

    pltpu.sync_copy(bias_hbm, bbuf)
    bv = bbuf[pl.ds(0, 16)]
    iota16 = lax.iota(jnp.int32, L) * 16

    def vbody(v, _):
        off = v * 16
        sa = wa[pl.ds(off, 16)]
        sc = wc[pl.ds(off, 16)]
        for p in range(1, NW):
            sa = sa + wa[pl.ds(p * WIN + off, 16)]
            sc = sc + wc[pl.ds(p * WIN + off, 16)]
        wa[pl.ds(off, 16)] = sa
        wc[pl.ds(off, 16)] = sc
        return 0
    lax.fori_loop(0, SEGW, vbody, 0)

    for g in range(SEGW // L):
        ta = jnp.zeros((L,), jnp.float32)
        tc = jnp.zeros((L,), jnp.float32)
        for l in range(L):
            idx = iota16 + (g * 256 + l)
            ta = ta + plsc.load_gather(wa, [idx])
            tc = tc + plsc.load_gather(wc, [idx])
        outv[pl.ds(g * 16, 16)] = ta / jnp.maximum(tc, 1.0) + bv
    pltpu.sync_copy(outv, out_hbm.at[pl.ds(wid * SEGW, SEGW)])


def kernel(x, batch, W, b):
    x1 = x.reshape(-1)
    bi = batch.astype(jnp.int32)
    wv = W.reshape(D).astype(jnp.float32)
    b16 = jnp.broadcast_to(b.astype(jnp.float32), (L,))
    a, c = _phase1(x1, bi, wv)
    return _phase2(a, c, b16)
